# Initial kernel scaffold; baseline (speedup 1.0000x reference)
import jax
import jax.numpy as jnp
from jax.experimental import pallas as pl


def kernel(latent_vec, node_hws, valid_nodes, segment_ids, spatial_inds):
    raise NotImplementedError("write your pallas kernel here")


# speedup vs baseline: 23.0662x; 23.0662x over previous
import jax
import jax.numpy as jnp
from jax import lax
from jax.experimental import pallas as pl
from jax.experimental.pallas import tpu as pltpu
from jax.experimental.pallas import tpu_sc as plsc

B, T, N, D = 8, 4, 1024, 64
H, W, P = 512, 512, 4096
BT = B * T
L = 16
CHUNKS = P // L


def _rsqrt(x):
    i = lax.bitcast_convert_type(x, jnp.int32)
    i = 0x5F3759DF - lax.shift_right_arithmetic(i, 1)
    y = lax.bitcast_convert_type(i, jnp.float32)
    for _ in range(3):
        y = y * (1.5 - 0.5 * x * y * y)
    return y


def _body(lat_h, hws_h, val_h, seg_h, hi_h, wi_h,
          dep_h, img_h, nrm_h, vv_h,
          lat_v, hws_v, val_v, hi_v, wi_v, idx_v, seg_v,
          dep_v, img_v, nrm_v, vv_v,
          sem_tab, sem_pix, sem_seg):
    cidx = lax.axis_index("c")
    sidx = lax.axis_index("s")
    bt = sidx * 2 + cidx

    cp_lat = pltpu.async_copy(lat_h.at[bt], lat_v, sem_tab)
    cp_hws = pltpu.async_copy(hws_h.at[bt], hws_v, sem_tab)
    cp_val = pltpu.async_copy(val_h.at[bt], val_v, sem_tab)
    cp_hi = pltpu.async_copy(hi_h.at[bt], hi_v, sem_pix)
    cp_wi = pltpu.async_copy(wi_h.at[bt], wi_v, sem_pix)
    cp_hi.wait()
    cp_wi.wait()

    base_img = bt * (H * W)

    def mk_idx(j, carry):
        o = j * L
        idx_v[pl.ds(o, L)] = hi_v[pl.ds(o, L)] * W + wi_v[pl.ds(o, L)] + base_img
        return carry

    lax.fori_loop(0, CHUNKS, mk_idx, 0)

    cp_seg = pltpu.async_copy(seg_h.at[idx_v], seg_v, sem_seg)
    cp_lat.wait()
    cp_hws.wait()
    cp_val.wait()
    cp_seg.wait()

    iota = lax.broadcasted_iota(jnp.int32, (L,), 0)

    def chunk(j, carry):
        o = j * L
        segj = seg_v[pl.ds(o, L)]
        vmask = (segj >= 0) & (segj < N)
        sg = lax.min(lax.max(segj, 0), N - 1)
        hf = hi_v[pl.ds(o, L)].astype(jnp.float32) * (2.0 / (H - 1)) - 1.0
        wf = wi_v[pl.ds(o, L)].astype(jnp.float32) * (2.0 / (W - 1)) - 1.0
        cen_h = plsc.load_gather(hws_v, [sg * 2])
        cen_w = plsc.load_gather(hws_v, [sg * 2 + 1])
        vn = plsc.load_gather(val_v, [sg])
        vv = jnp.where(vmask, vn, 0.0)
        dH = hf - cen_h
        dW = wf - cen_w
        d3 = dH * dH
        d4 = dH * dW
        d5 = dW * dW
        s64 = sg * D

        def acc(ch0, stride):
            r = plsc.load_gather(lat_v, [s64 + ch0])
            for i, dd in ((1, dH), (2, dW), (3, d3), (4, d4), (5, d5)):
                a = plsc.load_gather(lat_v, [s64 + (ch0 + i * stride)])
                r = r + a * dd
            return r

        dep = jnp.minimum(acc(0, 1) * vv, -0.1)
        dep_v[pl.ds(o, L)] = dep
        vv_v[pl.ds(o, L)] = vv

        sidx3 = iota * 3 + (o * 3)
        for ci in range(3):
            u = acc(6 + ci, 3) * vv
            plsc.store_scatter(img_v, [sidx3 + ci], jnp.clip(u, -100.0, 100.0))
        w0 = acc(24, 3) * vv
        w1 = acc(25, 3) * vv
        w2 = acc(26, 3) * vv
        r = _rsqrt(jnp.maximum(w0 * w0 + w1 * w1 + w2 * w2, 1e-12))
        plsc.store_scatter(nrm_v, [sidx3], w0 * r)
        plsc.store_scatter(nrm_v, [sidx3 + 1], w1 * r)
        plsc.store_scatter(nrm_v, [sidx3 + 2], w2 * r)
        return carry

    lax.fori_loop(0, CHUNKS, chunk, 0)

    pltpu.sync_copy(dep_v, dep_h.at[bt])
    pltpu.sync_copy(img_v, img_h.at[bt])
    pltpu.sync_copy(nrm_v, nrm_h.at[bt])
    pltpu.sync_copy(vv_v, vv_h.at[bt])


def kernel(latent_vec, node_hws, valid_nodes, segment_ids, spatial_inds):
    lat = latent_vec.reshape(BT, N * D)
    hws = node_hws.reshape(BT, N * 2)
    val = valid_nodes.reshape(BT, N)
    seg = segment_ids.reshape(BT * H * W)
    hi = spatial_inds[..., 0].reshape(BT, P)
    wi = spatial_inds[..., 1].reshape(BT, P)

    mesh = plsc.VectorSubcoreMesh(core_axis_name="c", subcore_axis_name="s",
                                  num_cores=2, num_subcores=16)
    f = pl.kernel(
        _body,
        out_type=(
            jax.ShapeDtypeStruct((BT, P), jnp.float32),
            jax.ShapeDtypeStruct((BT, 3 * P), jnp.float32),
            jax.ShapeDtypeStruct((BT, 3 * P), jnp.float32),
            jax.ShapeDtypeStruct((BT, P), jnp.float32),
        ),
        mesh=mesh,
        compiler_params=pltpu.CompilerParams(needs_layout_passes=False),
        scratch_types=[
            pltpu.VMEM((N * D,), jnp.float32),
            pltpu.VMEM((2 * N,), jnp.float32),
            pltpu.VMEM((N,), jnp.float32),
            pltpu.VMEM((P,), jnp.int32),
            pltpu.VMEM((P,), jnp.int32),
            pltpu.VMEM((P,), jnp.int32),
            pltpu.VMEM((P,), jnp.int32),
            pltpu.VMEM((P,), jnp.float32),
            pltpu.VMEM((3 * P,), jnp.float32),
            pltpu.VMEM((3 * P,), jnp.float32),
            pltpu.VMEM((P,), jnp.float32),
            pltpu.SemaphoreType.DMA,
            pltpu.SemaphoreType.DMA,
            pltpu.SemaphoreType.DMA,
        ],
    )
    dep, img, nrm, vv = f(lat, hws, val, seg, hi, wi)
    return (dep.reshape(B, T, P, 1),
            img.reshape(B, T, P, 3),
            nrm.reshape(B, T, P, 3),
            vv.reshape(B, T, P, 1))
